# baseline (speedup 1.0000x reference)
import functools
import math

import jax
import jax.numpy as jnp
from jax.experimental import pallas as pl

_BT = 1024
_G = 8


def _pe_block(o_ref, *, n, bt, c):
    i = pl.program_id(0)
    t0 = (i * bt).astype(jnp.float32)
    col = jax.lax.broadcasted_iota(jnp.int32, (_G, c), 1)
    colf = col.astype(jnp.float32)
    w = jnp.exp(colf * jnp.float32(-2.0 * math.log(10000.0) / c))
    phase = jnp.where((col & 1) == 1, jnp.float32(math.pi / 2.0), jnp.float32(0.0))
    r = jax.lax.broadcasted_iota(jnp.int32, (_G, c), 0).astype(jnp.float32)
    amp = jnp.float32(math.sqrt(c))
    theta0 = (t0 + r) * w + phase
    u0 = jnp.sin(theta0) * amp
    v0 = jnp.cos(theta0) * amp
    d = w * jnp.float32(_G)
    cd = jnp.cos(d)
    sd = jnp.sin(d)

    def body(k, carry):
        u, v = carry
        base = k * _G
        for b in range(n):
            o_ref[b, pl.ds(base, _G), :] = u
        return (u * cd + v * sd, v * cd - u * sd)

    jax.lax.fori_loop(0, bt // _G, body, (u0, v0))

    @pl.when(i == 0)
    def _():
        for b in range(n):
            o_ref[b, 0:1, :] = jnp.zeros((1, c), jnp.float32)


def kernel(inputs):
    n, t, c = inputs.shape
    body = functools.partial(_pe_block, n=n, bt=_BT, c=c)
    return pl.pallas_call(
        body,
        grid=(t // _BT,),
        out_specs=pl.BlockSpec((n, _BT, c), lambda i: (0, i, 0)),
        out_shape=jax.ShapeDtypeStruct((n, t, c), jnp.float32),
    )()


# speedup vs baseline: 1.0263x; 1.0263x over previous
import functools
import math

import jax
import jax.numpy as jnp
from jax.experimental import pallas as pl

_BT = 256
_G = 8


def _pe_block(o_ref, *, n, bt, c):
    i = pl.program_id(0)
    t0 = (i * bt).astype(jnp.float32)
    col = jax.lax.broadcasted_iota(jnp.int32, (_G, c), 1)
    colf = col.astype(jnp.float32)
    w = jnp.exp(colf * jnp.float32(-2.0 * math.log(10000.0) / c))
    phase = jnp.where((col & 1) == 1, jnp.float32(math.pi / 2.0), jnp.float32(0.0))
    r = jax.lax.broadcasted_iota(jnp.int32, (_G, c), 0).astype(jnp.float32)
    amp = jnp.float32(math.sqrt(c))
    theta0 = (t0 + r) * w + phase
    u0 = jnp.sin(theta0) * amp
    v0 = jnp.cos(theta0) * amp
    d = w * jnp.float32(_G)
    cd = jnp.cos(d)
    sd = jnp.sin(d)

    def body(k, carry):
        u, v = carry
        base = k * _G
        for b in range(n):
            o_ref[b, pl.ds(base, _G), :] = u
        return (u * cd + v * sd, v * cd - u * sd)

    jax.lax.fori_loop(0, bt // _G, body, (u0, v0))

    @pl.when(i == 0)
    def _():
        for b in range(n):
            o_ref[b, 0:1, :] = jnp.zeros((1, c), jnp.float32)


def kernel(inputs):
    n, t, c = inputs.shape
    body = functools.partial(_pe_block, n=n, bt=_BT, c=c)
    return pl.pallas_call(
        body,
        grid=(t // _BT,),
        out_specs=pl.BlockSpec((n, _BT, c), lambda i: (0, i, 0)),
        out_shape=jax.ShapeDtypeStruct((n, t, c), jnp.float32),
    )()


# speedup vs baseline: 1.0383x; 1.0117x over previous
import functools
import math

import jax
import jax.numpy as jnp
from jax.experimental import pallas as pl
from jax.experimental.pallas import tpu as pltpu

_BT = 512
_G = 8


def _pe_block(o_ref, *, n, bt, c):
    i = pl.program_id(0)
    t0 = (i * bt).astype(jnp.float32)
    col = jax.lax.broadcasted_iota(jnp.int32, (_G, c), 1)
    colf = col.astype(jnp.float32)
    w = jnp.exp(colf * jnp.float32(-2.0 * math.log(10000.0) / c))
    phase = jnp.where((col & 1) == 1, jnp.float32(math.pi / 2.0), jnp.float32(0.0))
    r = jax.lax.broadcasted_iota(jnp.int32, (_G, c), 0).astype(jnp.float32)
    amp = jnp.float32(math.sqrt(c))
    theta0 = (t0 + r) * w + phase
    u0 = jnp.sin(theta0) * amp
    v0 = jnp.cos(theta0) * amp
    d = w * jnp.float32(_G)
    cd = jnp.cos(d)
    sd = jnp.sin(d)

    def body(k, carry):
        u, v = carry
        base = k * _G
        for b in range(n):
            o_ref[b, pl.ds(base, _G), :] = u
        return (u * cd + v * sd, v * cd - u * sd)

    jax.lax.fori_loop(0, bt // _G, body, (u0, v0))

    @pl.when(i == 0)
    def _():
        for b in range(n):
            o_ref[b, 0:1, :] = jnp.zeros((1, c), jnp.float32)


def kernel(inputs):
    n, t, c = inputs.shape
    body = functools.partial(_pe_block, n=n, bt=_BT, c=c)
    return pl.pallas_call(
        body,
        grid=(t // _BT,),
        out_specs=pl.BlockSpec((n, _BT, c), lambda i: (0, i, 0)),
        out_shape=jax.ShapeDtypeStruct((n, t, c), jnp.float32),
        compiler_params=pltpu.CompilerParams(
            dimension_semantics=("parallel",),
        ),
    )()
